# baseline (speedup 1.0000x reference)
import functools
import math

import jax
import jax.numpy as jnp
from jax import lax
from jax.experimental import pallas as pl
from jax.experimental.pallas import tpu as pltpu
from jax.experimental.pallas import tpu_sc as plsc

_TOPK = 32
_RB = 64
_ILV = 4
_C = 512
_B = 32
_KTC = 16
_SC_OFF = _KTC * _C
_SC_ROWS = (_B - _KTC) * _C


def _scores_body(h_ref, w_ref, b_ref, scores_ref):
    h = h_ref[0]
    D = h.shape[1]
    C = h.shape[0]
    qk = jnp.dot(h, w_ref[...], preferred_element_type=jnp.float32) + b_ref[...]
    q = qk[:, :D]
    k = qk[:, D:]
    s = jax.lax.dot_general(q, k, (((1,), (1,)), ((), ())),
                            preferred_element_type=jnp.float32)
    s = s * (1.0 / math.sqrt(D))
    row = jax.lax.broadcasted_iota(jnp.int32, (C, C), 0)
    col = jax.lax.broadcasted_iota(jnp.int32, (C, C), 1)
    s = jnp.where(row == col, jnp.float32(-jnp.inf), s)
    scores_ref[0] = s


def _merge_step(r0v, r0i, r1v, r1i, rx, rxi):
    mk1 = r1v >= rx
    mv = jnp.where(mk1, r1v, rx)
    mi = jnp.where(mk1, r1i, rxi)
    mk2 = r0v >= mv
    uv = jnp.where(mk2, r0v, mv)
    ui = jnp.where(mk2, r0i, mi)
    lv = jnp.where(mk2, mv, r0v)
    li = jnp.where(mk2, mi, r0i)
    r0v, r0i = plsc.sort_key_val(uv, ui, descending=True)
    r1v, r1i = plsc.sort_key_val(lv, li, descending=True)
    return r0v, r0i, r1v, r1i


def _tc_topk_body(s_ref, idx_ref, val_ref):
    s = s_ref[0]
    C = s.shape[0]
    col = jax.lax.broadcasted_iota(jnp.int32, (C, C), 1)
    colf = col.astype(jnp.float32)
    neg_inf = jnp.float32(-jnp.inf)
    big = jnp.float32(C)
    cur = s
    m = jnp.max(cur, axis=1, keepdims=True)
    vals = []
    idxfs = []
    for t in range(_TOPK):
        idxf = jnp.min(jnp.where(cur == m, colf, big), axis=1, keepdims=True)
        vals.append(m)
        idxfs.append(idxf)
        if t < _TOPK - 1:
            cur = jnp.where(colf == idxf, neg_inf, cur)
            m = jnp.max(cur, axis=1, keepdims=True)
    val_ref[0] = jnp.concatenate(vals, axis=1)
    idx_ref[0] = jnp.concatenate(idxfs, axis=1).astype(jnp.int32)


def _sc_topk_body(scores_hbm, idx_hbm, val_hbm, vbuf, vob, iob):
    wid = lax.axis_index("s") * 2 + lax.axis_index("c")
    iota = lax.iota(jnp.int32, 16)
    rpw = _SC_ROWS // 32

    def _init_row(r):
        x0 = vbuf[r, pl.ds(0, 16)]
        x1 = vbuf[r, pl.ds(16, 16)]
        x0s, i0s = plsc.sort_key_val(x0, iota, descending=True)
        rx, rxi = plsc.sort_key_val(x1, iota + 16)
        mk = x0s >= rx
        uv = jnp.where(mk, x0s, rx)
        ui = jnp.where(mk, i0s, rxi)
        lv = jnp.where(mk, rx, x0s)
        li = jnp.where(mk, rxi, i0s)
        r0v, r0i = plsc.sort_key_val(uv, ui, descending=True)
        r1v, r1i = plsc.sort_key_val(lv, li, descending=True)
        return r0v, r0i, r1v, r1i

    def _store_row(r, st):
        r0v, r0i, r1v, r1i = st
        vob[r, pl.ds(0, 16)] = r0v
        vob[r, pl.ds(16, 16)] = r1v
        iob[r, pl.ds(0, 16)] = r0i
        iob[r, pl.ds(16, 16)] = r1i

    def do_rows(g, carry):
        rows = [g * _ILV + u for u in range(_ILV)]
        states = [_init_row(r) for r in rows]
        for j in range(2, 32):
            xs_all = []
            for u, r in enumerate(rows):
                x = vbuf[r, pl.ds(j * 16, 16)]
                xs_all.append(plsc.sort_key_val(x, iota + (16 * j)))
            for u in range(_ILV):
                xs, xis = xs_all[u]
                states[u] = _merge_step(*states[u], xs, xis)
        for u, r in enumerate(rows):
            _store_row(r, states[u])
        return carry

    def do_block(blk, carry):
        r0 = wid * rpw + blk * _RB
        pltpu.sync_copy(scores_hbm.at[pl.ds(_SC_OFF + r0, _RB)], vbuf)
        lax.fori_loop(0, _RB // _ILV, do_rows, 0, unroll=False)
        pltpu.sync_copy(vob, val_hbm.at[pl.ds(r0, _RB)])
        pltpu.sync_copy(iob, idx_hbm.at[pl.ds(r0, _RB)])
        return carry

    lax.fori_loop(0, rpw // _RB, do_block, 0, unroll=False)


def kernel(h, Wq, bq, Wk, bk):
    B, C, D = h.shape
    w = jnp.concatenate([Wq, Wk], axis=1)
    b = jnp.concatenate([bq, bk], axis=0)[None, :]
    scores = pl.pallas_call(
        _scores_body,
        grid=(B,),
        in_specs=[
            pl.BlockSpec((1, C, D), lambda i: (i, 0, 0)),
            pl.BlockSpec((D, 2 * D), lambda i: (0, 0)),
            pl.BlockSpec((1, 2 * D), lambda i: (0, 0)),
        ],
        out_specs=pl.BlockSpec((1, C, C), lambda i: (i, 0, 0)),
        out_shape=jax.ShapeDtypeStruct((B, C, C), jnp.float32),
        compiler_params=pltpu.CompilerParams(
            dimension_semantics=("parallel",),
        ),
    )(h, w, b)

    mesh = plsc.VectorSubcoreMesh(core_axis_name="c", subcore_axis_name="s")
    sc_call = functools.partial(
        pl.kernel,
        out_type=[
            jax.ShapeDtypeStruct((_SC_ROWS, _TOPK), jnp.int32),
            jax.ShapeDtypeStruct((_SC_ROWS, _TOPK), jnp.float32),
        ],
        mesh=mesh,
        scratch_types=[
            pltpu.VMEM((_RB, C), jnp.float32),
            pltpu.VMEM((_RB, _TOPK), jnp.float32),
            pltpu.VMEM((_RB, _TOPK), jnp.int32),
        ],
        compiler_params=pltpu.CompilerParams(needs_layout_passes=False),
    )(_sc_topk_body)
    idx_b, val_b = sc_call(scores.reshape(B * C, C))

    idx_a, val_a = pl.pallas_call(
        _tc_topk_body,
        grid=(_KTC,),
        in_specs=[pl.BlockSpec((1, C, C), lambda i: (i, 0, 0))],
        out_specs=[
            pl.BlockSpec((1, C, _TOPK), lambda i: (i, 0, 0)),
            pl.BlockSpec((1, C, _TOPK), lambda i: (i, 0, 0)),
        ],
        out_shape=[
            jax.ShapeDtypeStruct((_KTC, C, _TOPK), jnp.int32),
            jax.ShapeDtypeStruct((_KTC, C, _TOPK), jnp.float32),
        ],
        compiler_params=pltpu.CompilerParams(
            dimension_semantics=("parallel",),
        ),
    )(scores)

    idx = jnp.concatenate(
        [idx_a, idx_b.reshape(B - _KTC, C, _TOPK)], axis=0)
    val = jnp.concatenate(
        [val_a, val_b.reshape(B - _KTC, C, _TOPK)], axis=0)
    return (idx, val, scores)


# speedup vs baseline: 1.2937x; 1.2937x over previous
import functools
import math

import jax
import jax.numpy as jnp
from jax import lax
from jax.experimental import pallas as pl
from jax.experimental.pallas import tpu as pltpu
from jax.experimental.pallas import tpu_sc as plsc

_TOPK = 32
_RB = 64
_ILV = 4
_C = 512
_B = 32
_KTC = 10
_SC_OFF = _KTC * _C
_SC_ROWS = (_B - _KTC) * _C


def _scores_body(h_ref, w_ref, b_ref, scores_ref):
    h = h_ref[0]
    D = h.shape[1]
    C = h.shape[0]
    qk = jnp.dot(h, w_ref[...], preferred_element_type=jnp.float32) + b_ref[...]
    q = qk[:, :D]
    k = qk[:, D:]
    s = jax.lax.dot_general(q, k, (((1,), (1,)), ((), ())),
                            preferred_element_type=jnp.float32)
    s = s * (1.0 / math.sqrt(D))
    row = jax.lax.broadcasted_iota(jnp.int32, (C, C), 0)
    col = jax.lax.broadcasted_iota(jnp.int32, (C, C), 1)
    s = jnp.where(row == col, jnp.float32(-jnp.inf), s)
    scores_ref[0] = s


def _merge_step(r0v, r0i, r1v, r1i, rx, rxi):
    mk1 = r1v >= rx
    mv = jnp.where(mk1, r1v, rx)
    mi = jnp.where(mk1, r1i, rxi)
    mk2 = r0v >= mv
    uv = jnp.where(mk2, r0v, mv)
    ui = jnp.where(mk2, r0i, mi)
    lv = jnp.where(mk2, mv, r0v)
    li = jnp.where(mk2, mi, r0i)
    r0v, r0i = plsc.sort_key_val(uv, ui, descending=True)
    r1v, r1i = plsc.sort_key_val(lv, li, descending=True)
    return r0v, r0i, r1v, r1i


def _tc_topk_body(s_ref, idx_ref, val_ref):
    s = s_ref[0]
    C = s.shape[0]
    col = jax.lax.broadcasted_iota(jnp.int32, (C, C), 1)
    colf = col.astype(jnp.float32)
    neg_inf = jnp.float32(-jnp.inf)
    big = jnp.float32(C)
    cur = s
    m = jnp.max(cur, axis=1, keepdims=True)
    vals = []
    idxfs = []
    for t in range(_TOPK):
        idxf = jnp.min(jnp.where(cur == m, colf, big), axis=1, keepdims=True)
        vals.append(m)
        idxfs.append(idxf)
        if t < _TOPK - 1:
            cur = jnp.where(colf == idxf, neg_inf, cur)
            m = jnp.max(cur, axis=1, keepdims=True)
    val_ref[0] = jnp.concatenate(vals, axis=1)
    idx_ref[0] = jnp.concatenate(idxfs, axis=1).astype(jnp.int32)


def _sc_topk_body(scores_hbm, idx_hbm, val_hbm, vbuf, vob, iob):
    wid = lax.axis_index("s") * 2 + lax.axis_index("c")
    iota = lax.iota(jnp.int32, 16)
    rpw = _SC_ROWS // 32

    def _init_row(r):
        x0 = vbuf[r, pl.ds(0, 16)]
        x1 = vbuf[r, pl.ds(16, 16)]
        x0s, i0s = plsc.sort_key_val(x0, iota, descending=True)
        rx, rxi = plsc.sort_key_val(x1, iota + 16)
        mk = x0s >= rx
        uv = jnp.where(mk, x0s, rx)
        ui = jnp.where(mk, i0s, rxi)
        lv = jnp.where(mk, rx, x0s)
        li = jnp.where(mk, rxi, i0s)
        r0v, r0i = plsc.sort_key_val(uv, ui, descending=True)
        r1v, r1i = plsc.sort_key_val(lv, li, descending=True)
        return r0v, r0i, r1v, r1i

    def _store_row(r, st):
        r0v, r0i, r1v, r1i = st
        vob[r, pl.ds(0, 16)] = r0v
        vob[r, pl.ds(16, 16)] = r1v
        iob[r, pl.ds(0, 16)] = r0i
        iob[r, pl.ds(16, 16)] = r1i

    def do_rows(g, carry):
        rows = [g * _ILV + u for u in range(_ILV)]
        states = [_init_row(r) for r in rows]
        for j in range(2, 32):
            xs_all = []
            for u, r in enumerate(rows):
                x = vbuf[r, pl.ds(j * 16, 16)]
                xs_all.append(plsc.sort_key_val(x, iota + (16 * j)))
            for u in range(_ILV):
                xs, xis = xs_all[u]
                states[u] = _merge_step(*states[u], xs, xis)
        for u, r in enumerate(rows):
            _store_row(r, states[u])
        return carry

    def do_block(blk, carry):
        r0 = wid * rpw + blk * _RB
        pltpu.sync_copy(scores_hbm.at[pl.ds(_SC_OFF + r0, _RB)], vbuf)
        lax.fori_loop(0, _RB // _ILV, do_rows, 0, unroll=False)
        pltpu.sync_copy(vob, val_hbm.at[pl.ds(r0, _RB)])
        pltpu.sync_copy(iob, idx_hbm.at[pl.ds(r0, _RB)])
        return carry

    lax.fori_loop(0, rpw // _RB, do_block, 0, unroll=False)


def kernel(h, Wq, bq, Wk, bk):
    B, C, D = h.shape
    w = jnp.concatenate([Wq, Wk], axis=1)
    b = jnp.concatenate([bq, bk], axis=0)[None, :]
    scores = pl.pallas_call(
        _scores_body,
        grid=(B,),
        in_specs=[
            pl.BlockSpec((1, C, D), lambda i: (i, 0, 0)),
            pl.BlockSpec((D, 2 * D), lambda i: (0, 0)),
            pl.BlockSpec((1, 2 * D), lambda i: (0, 0)),
        ],
        out_specs=pl.BlockSpec((1, C, C), lambda i: (i, 0, 0)),
        out_shape=jax.ShapeDtypeStruct((B, C, C), jnp.float32),
        compiler_params=pltpu.CompilerParams(
            dimension_semantics=("parallel",),
        ),
    )(h, w, b)

    mesh = plsc.VectorSubcoreMesh(core_axis_name="c", subcore_axis_name="s")
    sc_call = functools.partial(
        pl.kernel,
        out_type=[
            jax.ShapeDtypeStruct((_SC_ROWS, _TOPK), jnp.int32),
            jax.ShapeDtypeStruct((_SC_ROWS, _TOPK), jnp.float32),
        ],
        mesh=mesh,
        scratch_types=[
            pltpu.VMEM((_RB, C), jnp.float32),
            pltpu.VMEM((_RB, _TOPK), jnp.float32),
            pltpu.VMEM((_RB, _TOPK), jnp.int32),
        ],
        compiler_params=pltpu.CompilerParams(needs_layout_passes=False),
    )(_sc_topk_body)
    idx_b, val_b = sc_call(scores.reshape(B * C, C))

    idx_a, val_a = pl.pallas_call(
        _tc_topk_body,
        grid=(_KTC,),
        in_specs=[pl.BlockSpec((1, C, C), lambda i: (i, 0, 0))],
        out_specs=[
            pl.BlockSpec((1, C, _TOPK), lambda i: (i, 0, 0)),
            pl.BlockSpec((1, C, _TOPK), lambda i: (i, 0, 0)),
        ],
        out_shape=[
            jax.ShapeDtypeStruct((_KTC, C, _TOPK), jnp.int32),
            jax.ShapeDtypeStruct((_KTC, C, _TOPK), jnp.float32),
        ],
        compiler_params=pltpu.CompilerParams(
            dimension_semantics=("parallel",),
        ),
    )(scores)

    idx = jnp.concatenate(
        [idx_a, idx_b.reshape(B - _KTC, C, _TOPK)], axis=0)
    val = jnp.concatenate(
        [val_a, val_b.reshape(B - _KTC, C, _TOPK)], axis=0)
    return (idx, val, scores)
